# baseline (speedup 1.0000x reference)
import functools

import jax
import jax.numpy as jnp
from jax import lax
from jax.experimental import pallas as pl
from jax.experimental.pallas import tpu as pltpu
from jax.experimental.pallas import tpu_sc as plsc

VOCAB = 1000000
DIM = 64
NC = 2
NS = 16
NW = NC * NS
LANES = 16

CHUNK = 256
SUB = 128
NSUB = CHUNK // SUB
NBUF = 5


def _make_sc_gather(batch):
    assert batch % (8 * NW) == 0
    per_w = batch // NW
    assert per_w % (CHUNK * NBUF) == 0
    nch = per_w // CHUNK

    mesh = plsc.VectorSubcoreMesh(core_axis_name="c", subcore_axis_name="s")

    @functools.partial(
        pl.kernel,
        mesh=mesh,
        compiler_params=pltpu.CompilerParams(
            use_tc_tiling_on_sc=False, needs_layout_passes=False),
        out_type=jax.ShapeDtypeStruct((batch, DIM), jnp.float32),
        scratch_types=[
            pltpu.VMEM((per_w,), jnp.int32),
            pltpu.VMEM((NBUF, CHUNK, DIM), jnp.float32),
            pltpu.VMEM((CHUNK + LANES,), jnp.int32),
        ] + [pltpu.SemaphoreType.DMA] * (2 * NBUF),
    )
    def grab(tab_hbm, idx_hbm, out_hbm, idx_v, rows_v, pos_v, *sems):
        gsems = sems[:NBUF]
        osems = sems[NBUF:]
        wid = lax.axis_index("s") * NC + lax.axis_index("c")
        base = wid * per_w

        pltpu.make_async_copy(idx_hbm.at[pl.ds(base, per_w)], idx_v, gsems[0]).start()
        pltpu.make_async_copy(idx_hbm.at[pl.ds(base, per_w)], idx_v, gsems[0]).wait()

        def fire_gathers(g, buf):
            off = g * CHUNK
            for s in range(NSUB):
                pltpu.make_async_copy(
                    tab_hbm.at[idx_v.at[pl.ds(off + s * SUB, SUB)]],
                    rows_v.at[buf, pl.ds(s * SUB, SUB), :],
                    gsems[buf],
                ).start()

        def drain_gathers(buf):
            for s in range(NSUB):
                pltpu.make_async_copy(
                    tab_hbm.at[idx_v.at[pl.ds(s * SUB, SUB)]],
                    rows_v.at[buf, pl.ds(s * SUB, SUB), :],
                    gsems[buf],
                ).wait()

        def out_copy(g, buf):
            return pltpu.make_async_copy(
                rows_v.at[buf],
                out_hbm.at[pl.ds(base + g * CHUNK, CHUNK)],
                osems[buf],
            )

        for b in range(NBUF - 1):
            fire_gathers(b, b)

        zeros16 = jnp.zeros((LANES,), jnp.float32)
        iota16 = lax.iota(jnp.int32, LANES)

        def round_body(g0, carry):
            for b in range(NBUF):
                g = g0 + b
                drain_gathers(b)

                @pl.when(g >= 1)
                def _():
                    out_copy(g - 1, (b - 1) % NBUF).wait()

                @pl.when(g + NBUF - 1 < nch)
                def _():
                    fire_gathers(g + NBUF - 1, (b - 1) % NBUF)

                off = g * CHUNK

                def scan_step(j, cnt):
                    v = idx_v[pl.ds(off + j * LANES, LANES)]
                    m = v == 0
                    ids = iota16 + j * LANES
                    s = m.astype(jnp.int32)
                    incl = plsc.cumsum(s)
                    plsc.store_scatter(pos_v, [cnt + incl - s], ids, mask=m)
                    return cnt + incl[LANES - 1]

                cnt = lax.fori_loop(0, CHUNK // LANES, scan_step, jnp.int32(0))

                def fix_step(i, fcarry):
                    p = pos_v[pl.ds(i, LANES)][0]
                    for c in range(DIM // LANES):
                        rows_v[b, p, pl.ds(c * LANES, LANES)] = zeros16
                    return fcarry

                lax.fori_loop(0, cnt, fix_step, 0)

                out_copy(g, b).start()
            return carry

        lax.fori_loop(0, nch // NBUF, lambda r, c: round_body(r * NBUF, c), 0)

        out_copy(nch - 1, (NBUF - 1) % NBUF).wait()

    return grab


def kernel(inputs, embeddings):
    batch = inputs.shape[0] * inputs.shape[1]
    idx_flat = inputs.reshape(batch)
    out = _make_sc_gather(batch)(embeddings, idx_flat)
    return out.reshape(inputs.shape[0], inputs.shape[1], DIM)
